# baseline (speedup 1.0000x reference)
import functools

import jax
import jax.numpy as jnp
from jax import lax
from jax.experimental import pallas as pl
from jax.experimental.pallas import tpu as pltpu
from jax.experimental.pallas import tpu_sc as plsc

EMB_DIM = 64
NUM_WORKERS = 32
NB = 16
IDX_PAD = 128
HIST_PAD = 56


def _make_gather(n_batch: int, hist: int):
    b_per_w = n_batch // NUM_WORKERS
    n_chunks = b_per_w // NB
    mesh = plsc.VectorSubcoreMesh(core_axis_name="c", subcore_axis_name="s")

    @functools.partial(
        pl.kernel,
        mesh=mesh,
        out_type=jax.ShapeDtypeStruct((n_batch, hist, EMB_DIM), jnp.float32),
        scratch_types=[
            pltpu.VMEM((NB * IDX_PAD,), jnp.int32),
            pltpu.VMEM((NB, HIST_PAD, EMB_DIM), jnp.float32),
            pltpu.SemaphoreType.DMA,
            pltpu.SemaphoreType.DMA,
        ],
        compiler_params=pltpu.CompilerParams(use_tc_tiling_on_sc=False),
    )
    def gather_kernel(table_hbm, idx_hbm, out_hbm, idx_v, rows_v, sem_g, sem_w):
        wid = lax.axis_index("s") * 2 + lax.axis_index("c")
        base = wid * b_per_w

        def body(c, carry):
            off = base + c * NB
            pltpu.sync_copy(
                idx_hbm.at[pl.ds(off * IDX_PAD, NB * IDX_PAD)], idx_v
            )
            for i in range(NB):
                pltpu.async_copy(
                    table_hbm.at[idx_v.at[pl.ds(i * IDX_PAD, HIST_PAD)]],
                    rows_v.at[i],
                    sem_g,
                )
            for i in range(NB):
                pltpu.make_async_copy(
                    table_hbm.at[idx_v.at[pl.ds(i * IDX_PAD, HIST_PAD)]],
                    rows_v.at[i],
                    sem_g,
                ).wait()
            for i in range(NB):
                pltpu.async_copy(
                    rows_v.at[i, pl.ds(0, hist)], out_hbm.at[off + i], sem_w
                )
            for i in range(NB):
                pltpu.make_async_copy(
                    rows_v.at[i, pl.ds(0, hist)], out_hbm.at[off + i], sem_w
                ).wait()
            return carry

        lax.fori_loop(0, n_chunks, body, 0)

    return gather_kernel


def kernel(batch, table):
    b, h = batch.shape
    batch_pad = jnp.pad(batch, ((0, 0), (0, IDX_PAD - h)))
    batch_flat = batch_pad.reshape(b * IDX_PAD)
    return _make_gather(b, h)(table, batch_flat)


# speedup vs baseline: 2.6310x; 2.6310x over previous
import functools

import jax
import jax.numpy as jnp
from jax import lax
from jax.experimental import pallas as pl
from jax.experimental.pallas import tpu as pltpu
from jax.experimental.pallas import tpu_sc as plsc

EMB_DIM = 64
NUM_WORKERS = 32
NB = 16
IDX_PAD = 128
HIST_PAD = 56


def _make_gather(n_batch: int, hist: int):
    b_per_w = n_batch // NUM_WORKERS
    n_chunks = b_per_w // NB
    mesh = plsc.VectorSubcoreMesh(core_axis_name="c", subcore_axis_name="s")

    @functools.partial(
        pl.kernel,
        mesh=mesh,
        out_type=jax.ShapeDtypeStruct((n_batch, hist, EMB_DIM), jnp.float32),
        scratch_types=[
            pltpu.VMEM((NB * IDX_PAD,), jnp.int32),
            pltpu.VMEM((NB, HIST_PAD, EMB_DIM), jnp.float32),
            pltpu.SemaphoreType.DMA,
            pltpu.SemaphoreType.DMA,
        ],
        compiler_params=pltpu.CompilerParams(use_tc_tiling_on_sc=False),
    )
    def gather_kernel(table_hbm, idx_hbm, out_hbm, idx_v, rows_v, sem_g, sem_w):
        wid = lax.axis_index("s") * 2 + lax.axis_index("c")
        base = wid * b_per_w

        def body(c, carry):
            off = base + c * NB
            pltpu.sync_copy(
                idx_hbm.at[pl.ds(off * IDX_PAD, NB * IDX_PAD)], idx_v
            )
            for i in range(NB):
                pltpu.async_copy(
                    table_hbm.at[idx_v.at[pl.ds(i * IDX_PAD, HIST_PAD)]],
                    rows_v.at[i],
                    sem_g,
                )
            for i in range(NB):
                pltpu.make_async_copy(
                    table_hbm.at[idx_v.at[pl.ds(i * IDX_PAD, HIST_PAD)]],
                    rows_v.at[i],
                    sem_g,
                ).wait()
            for i in range(NB):
                pltpu.async_copy(
                    rows_v.at[i, pl.ds(0, hist)], out_hbm.at[off + i], sem_w
                )
            for i in range(NB):
                pltpu.make_async_copy(
                    rows_v.at[i, pl.ds(0, hist)], out_hbm.at[off + i], sem_w
                ).wait()
            return carry

        lax.fori_loop(0, n_chunks, body, 0)

    return gather_kernel


def kernel(batch, table):
    b, h = batch.shape
    filler = (
        jnp.arange(b, dtype=jnp.int32)[:, None] * 1009
        + jnp.arange(IDX_PAD - h, dtype=jnp.int32)[None, :] * 131071
    ) % table.shape[0]
    batch_pad = jnp.concatenate([batch, filler], axis=1)
    batch_flat = batch_pad.reshape(b * IDX_PAD)
    return _make_gather(b, h)(table, batch_flat)
